# baseline (speedup 1.0000x reference)
import numpy as np

import jax
import jax.numpy as jnp
from jax import lax
from jax.experimental import pallas as pl
from jax.experimental.pallas import tpu as pltpu

_NC = 10
_S = 8
_HW0 = 768
_HW1 = 256
_W0 = _S * _HW0
_W1 = _S * _HW1
_HALF = _W0 // 2


def _fused_kernel(x_ref, w1_ref, g1_ref, w2_ref, sh2_ref,
                  w3_ref, b3_ref, g2_ref, e2_ref, o_ref,
                  x9_ref, rz_ref):
    f32 = jnp.float32
    bf16 = jnp.bfloat16

    def shifted(v, s, width):
        if s == 0:
            return v
        return pltpu.roll(v, shift=width - s, axis=1)

    def pool_hw(v, stride, width):
        t = jnp.maximum(v, shifted(v, 1, width))
        return jnp.maximum(t, shifted(t, stride, width))

    xin = x_ref[...]
    for k in range(9):
        kh, kw = divmod(k, 3)
        x9_ref[27 * k:27 * (k + 1), :] = (
            shifted(xin, kh * 27 + kw, _W0).astype(bf16))
    x9_ref[243:244, :] = jnp.ones((1, _W0), bf16)

    for s in range(_S):
        xh = x9_ref[:, pl.ds(_HW0 * s, _HW0)]
        y = jnp.dot(w1_ref[...], xh, preferred_element_type=f32)
        y4 = y.reshape(12, 2, 32, _HW0)
        m = jnp.maximum(jnp.maximum(y4[:, 0], y4[:, 1]), 0.0)
        m = pool_hw(m.reshape(384, _HW0), 27, _HW0).astype(bf16)
        for k in range(9):
            zk = jnp.dot(m, g1_ref[:, 256 * k:256 * (k + 1)],
                         preferred_element_type=f32)
            rz_ref[384 * k:384 * (k + 1),
                   _HW1 * s:_HW1 * (s + 1)] = zk.astype(bf16)

    def stage2_body(i, acc):
        patch = jnp.concatenate(
            [rz_ref[pl.ds(384 * k + 64 * i, 128), :] for k in range(9)],
            axis=0)
        y = jnp.dot(w2_ref[...], patch, preferred_element_type=f32)
        y = y + sh2_ref[...]
        m = jnp.maximum(jnp.maximum(y[0:64, :], y[64:128, :]), 0.0)
        m = pool_hw(m, 12, _W1).astype(bf16)
        mc = jnp.dot(m, g2_ref[...], preferred_element_type=f32)
        w3d = w3_ref[i]
        rows = [jnp.sum(w3d[k] * mc, axis=0, keepdims=True)
                for k in range(_NC)]
        return acc + jnp.concatenate(rows, axis=0)

    logits = lax.fori_loop(0, 5, stage2_body, jnp.zeros((_NC, 256), f32))
    o_ref[...] = (jnp.dot(logits, e2_ref[...], preferred_element_type=f32)
                  + b3_ref[...])


def _pool2_constants():
    g2 = np.zeros((_W1, _S * 32), np.float32)
    for s in range(_S):
        for hh in range(5):
            for ww in range(5):
                g2[s * _HW1 + 24 * hh + 2 * ww, s * 32 + 5 * hh + ww] = 1.0
    e2 = np.zeros((_S * 32, _S), np.float32)
    for s in range(_S):
        e2[s * 32:(s + 1) * 32, s] = 1.0
    return jnp.asarray(g2, jnp.bfloat16), jnp.asarray(e2)


def _toeplitz_w1(w1f):
    d = np.repeat(np.arange(24), 27)
    k9 = np.tile(np.repeat(np.arange(9), 3), 24)
    kd = np.tile(np.arange(3), 24 * 9)
    vals = w1f.T[kd * 9 + k9]
    w4 = jnp.zeros((24, 9, 27, 32), jnp.float32)
    w4 = w4.at[d, k9, d + kd].set(vals)
    return w4.transpose(0, 3, 1, 2).reshape(768, 243).astype(jnp.bfloat16)


def kernel(x, w1f, shift1, g1, w2p, shift2, w3s, b3):
    b, r = x.shape[0], x.shape[1]
    n = b * r
    g = n // _S

    xk = x.reshape(n, 27, 729).astype(jnp.float32)
    xk = jnp.pad(xk, ((0, 0), (0, 0), (0, _HW0 - 729)))
    xk = xk.reshape(g, _S, 27, _HW0).transpose(0, 2, 1, 3).reshape(g, 27, _W0)

    sh1t = jnp.tile(shift1, (24, 1))
    w1t = jnp.concatenate([_toeplitz_w1(w1f).astype(jnp.float32), sh1t],
                          axis=1).astype(jnp.bfloat16)
    g1s = jnp.concatenate(
        [jnp.pad(g1[:, 12 * (k // 3) + k % 3:],
                 ((0, 0), (0, 12 * (k // 3) + k % 3)))
         for k in range(9)], axis=1).astype(jnp.bfloat16)
    w2r = w2p.reshape(64, 3, 3, 3, 32).transpose(0, 2, 3, 1, 4)
    w2r = w2r.reshape(64, 9, 3, 32)
    w2q = jnp.zeros((2, 64, 9, 4, 32), jnp.float32)
    for a in range(2):
        for e in range(4):
            if 0 <= e - a <= 2:
                w2q = w2q.at[a, :, :, e, :].set(w2r[:, :, e - a, :])
    w2b = w2q.reshape(128, 1152).astype(jnp.bfloat16)
    sh2t = jnp.tile(shift2, (2, 1))

    cols = np.array([24 * hh + 2 * ww for hh in range(5) for ww in range(5)])
    w3c = w3s[:, :, :, cols]
    w3c = jnp.pad(w3c, ((0, 0), (0, 0), (0, 0), (0, 7)))
    w3t = jnp.tile(w3c, (1, 1, 1, _S))

    g2m, e2m = _pool2_constants()
    b3t = b3.reshape(_NC, 1)

    out = pl.pallas_call(
        _fused_kernel,
        out_shape=jax.ShapeDtypeStruct((g, _NC, _S), jnp.float32),
        grid=(g,),
        in_specs=[
            pl.BlockSpec((None, 27, _W0), lambda i: (i, 0, 0)),
            pl.BlockSpec((768, 244), lambda i: (0, 0)),
            pl.BlockSpec((768, 9 * _HW1), lambda i: (0, 0)),
            pl.BlockSpec((128, 1152), lambda i: (0, 0)),
            pl.BlockSpec((128, 1), lambda i: (0, 0)),
            pl.BlockSpec((5, _NC, 64, 256), lambda i: (0, 0, 0, 0)),
            pl.BlockSpec((_NC, 1), lambda i: (0, 0)),
            pl.BlockSpec((_W1, _S * 32), lambda i: (0, 0)),
            pl.BlockSpec((_S * 32, _S), lambda i: (0, 0)),
        ],
        out_specs=pl.BlockSpec((None, _NC, _S), lambda i: (i, 0, 0)),
        scratch_shapes=[
            pltpu.VMEM((244, _W0), jnp.bfloat16),
            pltpu.VMEM((3456, _W1), jnp.bfloat16),
        ],
        compiler_params=pltpu.CompilerParams(
            dimension_semantics=("parallel",),
            vmem_limit_bytes=56 * 1024 * 1024,
        ),
    )(xk, w1t, g1s, w2b, sh2t, w3t, b3t, g2m, e2m)

    return out.transpose(0, 2, 1).reshape(b, r, _NC)


# speedup vs baseline: 1.2590x; 1.2590x over previous
import numpy as np

import jax
import jax.numpy as jnp
from jax import lax
from jax.experimental import pallas as pl
from jax.experimental.pallas import tpu as pltpu

_NC = 10
_S = 8
_HW0 = 768
_HW1 = 256
_W0 = _S * _HW0
_W1 = _S * _HW1
_HALF = _W0 // 2


def _fused_kernel(x_ref, w1_ref, g1_ref, w2_ref, sh2_ref,
                  w3_ref, b3_ref, g2_ref, e2_ref, o_ref,
                  x9_ref, z_ref, rz_ref):
    f32 = jnp.float32
    bf16 = jnp.bfloat16

    def shifted(v, s, width):
        if s == 0:
            return v
        return pltpu.roll(v, shift=width - s, axis=1)

    def pool_hw(v, stride, width):
        t = jnp.maximum(v, shifted(v, 1, width))
        return jnp.maximum(t, shifted(t, stride, width))

    xin = x_ref[...]
    for k in range(9):
        kh, kw = divmod(k, 3)
        x9_ref[27 * k:27 * (k + 1), :] = (
            shifted(xin, kh * 27 + kw, _W0).astype(bf16))
    x9_ref[243:244, :] = jnp.ones((1, _W0), bf16)

    for s in range(_S):
        xh = x9_ref[:, pl.ds(_HW0 * s, _HW0)]
        y = jnp.dot(w1_ref[...], xh, preferred_element_type=f32)
        y4 = y.reshape(12, 2, 32, _HW0)
        m = jnp.maximum(jnp.maximum(y4[:, 0], y4[:, 1]), 0.0)
        m = pool_hw(m.reshape(384, _HW0), 27, _HW0).astype(bf16)
        zs = jnp.dot(m, g1_ref[...], preferred_element_type=f32)
        z_ref[:, pl.ds(_HW1 * s, _HW1)] = zs

    zall = z_ref[...]
    for k in range(9):
        kh, kw = divmod(k, 3)
        rz_ref[384 * k:384 * (k + 1), :] = (
            shifted(zall, kh * 12 + kw, _W1).astype(bf16))

    def stage2_body(i, acc):
        patch = jnp.concatenate(
            [rz_ref[pl.ds(384 * k + 64 * i, 128), :] for k in range(9)],
            axis=0)
        y = jnp.dot(w2_ref[...], patch, preferred_element_type=f32)
        y = y + sh2_ref[...]
        m = jnp.maximum(jnp.maximum(y[0:64, :], y[64:128, :]), 0.0)
        m = pool_hw(m, 12, _W1).astype(bf16)
        mc = jnp.dot(m, g2_ref[...], preferred_element_type=f32)
        w3d = w3_ref[i]
        rows = [jnp.sum(w3d[k] * mc, axis=0, keepdims=True)
                for k in range(_NC)]
        return acc + jnp.concatenate(rows, axis=0)

    logits = lax.fori_loop(0, 5, stage2_body, jnp.zeros((_NC, 256), f32))
    o_ref[...] = (jnp.dot(logits, e2_ref[...], preferred_element_type=f32)
                  + b3_ref[...])


def _pool2_constants():
    g2 = np.zeros((_W1, _S * 32), np.float32)
    for s in range(_S):
        for hh in range(5):
            for ww in range(5):
                g2[s * _HW1 + 24 * hh + 2 * ww, s * 32 + 5 * hh + ww] = 1.0
    e2 = np.zeros((_S * 32, _S), np.float32)
    for s in range(_S):
        e2[s * 32:(s + 1) * 32, s] = 1.0
    return jnp.asarray(g2, jnp.bfloat16), jnp.asarray(e2)


def _toeplitz_w1(w1f):
    d = np.repeat(np.arange(24), 27)
    k9 = np.tile(np.repeat(np.arange(9), 3), 24)
    kd = np.tile(np.arange(3), 24 * 9)
    vals = w1f.T[kd * 9 + k9]
    w4 = jnp.zeros((24, 9, 27, 32), jnp.float32)
    w4 = w4.at[d, k9, d + kd].set(vals)
    return w4.transpose(0, 3, 1, 2).reshape(768, 243).astype(jnp.bfloat16)


def kernel(x, w1f, shift1, g1, w2p, shift2, w3s, b3):
    b, r = x.shape[0], x.shape[1]
    n = b * r
    g = n // _S

    xk = x.reshape(n, 27, 729).astype(jnp.float32)
    xk = jnp.pad(xk, ((0, 0), (0, 0), (0, _HW0 - 729)))
    xk = xk.reshape(g, _S, 27, _HW0).transpose(0, 2, 1, 3).reshape(g, 27, _W0)

    sh1t = jnp.tile(shift1, (24, 1))
    w1t = jnp.concatenate([_toeplitz_w1(w1f).astype(jnp.float32), sh1t],
                          axis=1).astype(jnp.bfloat16)
    g1b = g1.astype(jnp.bfloat16)
    w2r = w2p.reshape(64, 3, 3, 3, 32).transpose(0, 2, 3, 1, 4)
    w2r = w2r.reshape(64, 9, 3, 32)
    w2q = jnp.zeros((2, 64, 9, 4, 32), jnp.float32)
    for a in range(2):
        for e in range(4):
            if 0 <= e - a <= 2:
                w2q = w2q.at[a, :, :, e, :].set(w2r[:, :, e - a, :])
    w2b = w2q.reshape(128, 1152).astype(jnp.bfloat16)
    sh2t = jnp.tile(shift2, (2, 1))

    cols = np.array([24 * hh + 2 * ww for hh in range(5) for ww in range(5)])
    w3c = w3s[:, :, :, cols]
    w3c = jnp.pad(w3c, ((0, 0), (0, 0), (0, 0), (0, 7)))
    w3t = jnp.tile(w3c, (1, 1, 1, _S))

    g2m, e2m = _pool2_constants()
    b3t = b3.reshape(_NC, 1)

    out = pl.pallas_call(
        _fused_kernel,
        out_shape=jax.ShapeDtypeStruct((g, _NC, _S), jnp.float32),
        grid=(g,),
        in_specs=[
            pl.BlockSpec((None, 27, _W0), lambda i: (i, 0, 0)),
            pl.BlockSpec((768, 244), lambda i: (0, 0)),
            pl.BlockSpec((_HW0, _HW1), lambda i: (0, 0)),
            pl.BlockSpec((128, 1152), lambda i: (0, 0)),
            pl.BlockSpec((128, 1), lambda i: (0, 0)),
            pl.BlockSpec((5, _NC, 64, 256), lambda i: (0, 0, 0, 0)),
            pl.BlockSpec((_NC, 1), lambda i: (0, 0)),
            pl.BlockSpec((_W1, _S * 32), lambda i: (0, 0)),
            pl.BlockSpec((_S * 32, _S), lambda i: (0, 0)),
        ],
        out_specs=pl.BlockSpec((None, _NC, _S), lambda i: (i, 0, 0)),
        scratch_shapes=[
            pltpu.VMEM((244, _W0), jnp.bfloat16),
            pltpu.VMEM((384, _W1), jnp.float32),
            pltpu.VMEM((3456, _W1), jnp.bfloat16),
        ],
        compiler_params=pltpu.CompilerParams(
            dimension_semantics=("parallel",),
            vmem_limit_bytes=56 * 1024 * 1024,
        ),
    )(xk, w1t, g1b, w2b, sh2t, w3t, b3t, g2m, e2m)

    return out.transpose(0, 2, 1).reshape(b, r, _NC)
